# baseline (speedup 1.0000x reference)
import functools

import jax
import jax.numpy as jnp
from jax import lax
from jax.experimental import pallas as pl
from jax.experimental.pallas import tpu as pltpu
from jax.experimental.pallas import tpu_sc as plsc

_CH = 8


def _body(inp_ref, dir_ref, prob_ref, out_ref):
    _, H, W = out_ref.shape
    n = H // _CH

    jj = jax.lax.broadcasted_iota(jnp.int32, (_CH, W), 1)
    col_l, col_r = jj >= 1, jj < W - 1
    fill_col = jnp.full((_CH, 1), -1, jnp.int32)
    fill_row = jnp.full((1, W), -1, jnp.int32)
    true2 = jnp.full((_CH, W), True)

    def colL(x):
        return jnp.concatenate([x[:, 1:2], x[:, :-1]], axis=1)

    def colR(x):
        return jnp.concatenate([x[:, 1:], x[:, -2:-1]], axis=1)

    def colLm(x):
        return jnp.concatenate([fill_col, x[:, :-1]], axis=1)

    def colRm(x):
        return jnp.concatenate([x[:, 1:], fill_col], axis=1)

    def ld(k):
        s = slice(k * _CH, (k + 1) * _CH)
        a = inp_ref[0, s, :]
        e = jnp.where(prob_ref[0, s, :] <= 20, dir_ref[0, s, :], -1)
        return a, e

    a_p = e_p = a_n = e_n = None
    a_c, e_c = ld(0)
    for k in range(n):
        if k + 1 < n:
            a_n, e_n = ld(k + 1)
        if k == 0:
            up = jnp.concatenate([a_c[1:2], a_c[:_CH - 1]], axis=0)
            eu = jnp.concatenate([fill_row, e_c[:_CH - 1]], axis=0)
        else:
            up = jnp.concatenate([a_p[_CH - 1:], a_c[:_CH - 1]], axis=0)
            eu = jnp.concatenate([e_p[_CH - 1:], e_c[:_CH - 1]], axis=0)
        if k == n - 1:
            dn = jnp.concatenate([a_c[1:], a_c[_CH - 2:_CH - 1]], axis=0)
            ed = jnp.concatenate([e_c[1:], fill_row], axis=0)
        else:
            dn = jnp.concatenate([a_c[1:], a_n[:1]], axis=0)
            ed = jnp.concatenate([e_c[1:], e_n[:1]], axis=0)

        rs = up + a_c + dn
        avg = (colL(rs) + rs + colR(rs)) * (1.0 / 9.0)

        si = {0: colR(dn), 1: dn, 2: colL(dn), 3: colR(a_c),
              5: colL(a_c), 6: colR(up), 7: up}
        se = {0: colRm(ed), 1: ed, 2: colLm(ed), 3: colRm(e_c),
              5: colLm(e_c), 6: colRm(eu), 7: eu}

        if k == 0:
            ii = jax.lax.broadcasted_iota(jnp.int32, (_CH, W), 0)
            row_up = ii >= 1
        else:
            row_up = true2
        if k == n - 1:
            ii = jax.lax.broadcasted_iota(jnp.int32, (_CH, W), 0)
            row_dn = ii < _CH - 1
        else:
            row_dn = true2
        inb = {0: row_up & col_l, 1: row_up, 2: row_up & col_r, 3: col_l,
               5: col_r, 6: row_dn & col_l, 7: row_dn}

        x = a_c
        for d in range(8):
            if d == 4:
                x = jnp.where(e_c == 4, avg, x)
                continue
            x = jnp.where(se[d] == d, si[d], x)
            x = jnp.where((e_c == d) & inb[d], avg, x)
        out_ref[0, k * _CH:(k + 1) * _CH, :] = x

        a_p, e_p = a_c, e_c
        a_c, e_c = a_n, e_n


def _tc_call(a3, d3, p3, n_tc=None, interpret=False):
    N, H, W = a3.shape
    if n_tc is None:
        n_tc = N
    spec = pl.BlockSpec((1, H, W), lambda i: (i, 0, 0))
    return pl.pallas_call(
        _body,
        grid=(n_tc,),
        in_specs=[spec, spec, spec],
        out_specs=spec,
        out_shape=jax.ShapeDtypeStruct((N, H, W), a3.dtype),
        compiler_params=None if interpret else pltpu.CompilerParams(
            has_side_effects=False),
        interpret=interpret,
    )(a3, d3, p3)


_L = 16
_NW = 32
_R = 56
_RS = 72
_HW = 224


def _sc_body(inp_hbm, dir_hbm, prob_hbm, out_hbm, inp_b, e_b, p_b, out_b):
    H = W = _HW
    nj = W // _L
    n0 = inp_hbm.shape[0] - out_hbm.shape[0]
    ipw = out_hbm.shape[0] // _NW
    wid = lax.axis_index("s") * 2 + lax.axis_index("c")
    lane = lax.broadcasted_iota(jnp.int32, (_L,), 0)
    col_l = lane >= 1
    col_r = lane < _L - 1
    ninth = jnp.full((_L,), 1.0 / 9.0, jnp.float32)

    def load_row(lr, j):
        c0 = j * _L
        iC = inp_b[lr, pl.ds(c0, _L)]
        eC = e_b[lr, pl.ds(c0, _L)]
        rows = jnp.full((_L,), lr, jnp.int32)
        if j == 0:
            iL = plsc.load_gather(inp_b, [rows, jnp.where(lane == 0, 1, lane - 1)])
            eL = plsc.load_gather(e_b, [rows, jnp.maximum(lane - 1, 0)])
        else:
            iL = inp_b[lr, pl.ds(c0 - 1, _L)]
            eL = e_b[lr, pl.ds(c0 - 1, _L)]
        if j == nj - 1:
            iR = plsc.load_gather(
                inp_b, [rows, jnp.where(lane == _L - 1, W - 2, c0 + 1 + lane)])
            eR = plsc.load_gather(
                e_b, [rows, jnp.minimum(c0 + 1 + lane, W - 1)])
        else:
            iR = inp_b[lr, pl.ds(c0 + 1, _L)]
            eR = e_b[lr, pl.ds(c0 + 1, _L)]
        return iL, iC, iR, eL, eC, eR

    def img_body(t, _):
        img = n0 + wid * ipw + t

        def blk_body(b, _):
            g0 = b * _R
            w0 = jnp.minimum(jnp.maximum((g0 - 1) // 8 * 8, 0), H - _RS)
            w0 = pl.multiple_of(w0, 8)
            pltpu.sync_copy(inp_hbm.at[img, pl.ds(w0, _RS), :], inp_b)
            pltpu.sync_copy(dir_hbm.at[img, pl.ds(w0, _RS), :], e_b)
            pltpu.sync_copy(prob_hbm.at[img, pl.ds(w0, _RS), :], p_b)

            st = jnp.maximum(g0 - 1, 0) - w0

            def e_row(lr, _):
                for j in range(nj):
                    s = pl.ds(j * _L, _L)
                    e_b[lr, s] = jnp.where(p_b[lr, s] <= 20, e_b[lr, s], -1)
                return 0
            lax.fori_loop(st, st + _R + 2, e_row, 0)

            for j in range(nj):
                jl = col_l if j == 0 else None
                jr = col_r if j == nj - 1 else None

                def step(gl, carry, j=j, jl=jl, jr=jr, g0=g0, w0=w0):
                    (iuL, iuC, iuR, euL, euC, euR,
                     icL, icC, icR, ecL, ecC, ecR) = carry
                    g = g0 + gl
                    dr = jnp.where(g == H - 1, H - 2, g + 1)
                    idL, idC, idR, edL, edC, edR = load_row(dr - w0, j)
                    gv = jnp.full((_L,), g, jnp.int32)
                    row_up = gv > 0
                    row_dn = gv < H - 1

                    avg = (iuL + iuC + iuR + icL + icC + icR
                           + idL + idC + idR) * ninth

                    def m(c, extra):
                        for e_ in extra:
                            if e_ is not None:
                                c = c & e_
                        return c

                    x = icC
                    x = jnp.where(m((edR == 0), [row_dn, jr]), idR, x)
                    x = jnp.where(m((ecC == 0), [row_up, jl]), avg, x)
                    x = jnp.where(m((edC == 1), [row_dn]), idC, x)
                    x = jnp.where(m((ecC == 1), [row_up]), avg, x)
                    x = jnp.where(m((edL == 2), [row_dn, jl]), idL, x)
                    x = jnp.where(m((ecC == 2), [row_up, jr]), avg, x)
                    x = jnp.where(m((ecR == 3), [jr]), icR, x)
                    x = jnp.where(m((ecC == 3), [jl]), avg, x)
                    x = jnp.where((ecC == 4), avg, x)
                    x = jnp.where(m((ecL == 5), [jl]), icL, x)
                    x = jnp.where(m((ecC == 5), [jr]), avg, x)
                    x = jnp.where(m((euR == 6), [row_up, jr]), iuR, x)
                    x = jnp.where(m((ecC == 6), [row_dn, jl]), avg, x)
                    x = jnp.where(m((euC == 7), [row_up]), iuC, x)
                    x = jnp.where(m((ecC == 7), [row_dn]), avg, x)
                    out_b[gl, pl.ds(j * _L, _L)] = x

                    return (icL, icC, icR, ecL, ecC, ecR,
                            idL, idC, idR, edL, edC, edR)

                pm1 = jnp.where(g0 == 0, 1, g0 - 1)
                carry0 = load_row(pm1 - w0, j) + load_row(g0 - w0, j)
                lax.fori_loop(0, _R, step, carry0)

            pltpu.sync_copy(out_b, out_hbm.at[img - n0, pl.ds(g0, _R), :])
            return 0

        lax.fori_loop(0, H // _R, blk_body, 0)
        return 0

    lax.fori_loop(0, ipw, img_body, 0)


def _sc_call(a3, d3, p3, n_sc):
    N, H, W = a3.shape
    mesh = plsc.VectorSubcoreMesh(core_axis_name="c", subcore_axis_name="s",
                                  num_cores=2, num_subcores=16)
    f = pl.kernel(
        _sc_body,
        out_type=jax.ShapeDtypeStruct((n_sc, H, W), jnp.float32),
        mesh=mesh,
        compiler_params=pltpu.CompilerParams(use_tc_tiling_on_sc=False, needs_layout_passes=False,
            has_side_effects=False),
        scratch_types=[
            pltpu.VMEM((_RS, W), jnp.float32),
            pltpu.VMEM((_RS, W), jnp.int32),
            pltpu.VMEM((_RS, W), jnp.int32),
            pltpu.VMEM((_R, W), jnp.float32),
        ],
    )
    return f(a3, d3, p3)


_N_SC = 64


@functools.partial(jax.jit, static_argnames=("interpret",))
def kernel(inp, direction, prob, interpret=False):
    B, C, H, W = inp.shape
    N = B * C
    a3 = inp.reshape(N, H, W)
    d3 = direction.reshape(N, H, W)
    p3 = prob.reshape(N, H, W)
    if interpret or _N_SC == 0:
        return _tc_call(a3, d3, p3, interpret=interpret).reshape(B, C, H, W)
    n_tc = N - _N_SC
    out_sc = _sc_call(a3, d3, p3, _N_SC)
    out_tc = _tc_call(a3, d3, p3, n_tc=n_tc)
    out = lax.dynamic_update_slice(out_tc, out_sc, (n_tc, 0, 0))
    return out.reshape(B, C, H, W)


# speedup vs baseline: 1.7059x; 1.7059x over previous
import functools

import jax
import jax.numpy as jnp
from jax.experimental import pallas as pl

_CH = 8


def _body(inp_ref, dir_ref, prob_ref, out_ref):
    _, H, W = out_ref.shape
    n = H // _CH

    jj = jax.lax.broadcasted_iota(jnp.int32, (_CH, W), 1)
    col_l, col_r = jj >= 1, jj < W - 1
    fill_col = jnp.full((_CH, 1), -1, jnp.int32)
    fill_row = jnp.full((1, W), -1, jnp.int32)
    true2 = jnp.full((_CH, W), True)

    def colL(x):
        return jnp.concatenate([x[:, 1:2], x[:, :-1]], axis=1)

    def colR(x):
        return jnp.concatenate([x[:, 1:], x[:, -2:-1]], axis=1)

    def colLm(x):
        return jnp.concatenate([fill_col, x[:, :-1]], axis=1)

    def colRm(x):
        return jnp.concatenate([x[:, 1:], fill_col], axis=1)

    def ld(k):
        s = slice(k * _CH, (k + 1) * _CH)
        a = inp_ref[0, s, :]
        e = jnp.where(prob_ref[0, s, :] <= 20, dir_ref[0, s, :], -1)
        return a, e

    a_p = e_p = a_n = e_n = None
    a_c, e_c = ld(0)
    for k in range(n):
        if k + 1 < n:
            a_n, e_n = ld(k + 1)
        if k == 0:
            up = jnp.concatenate([a_c[1:2], a_c[:_CH - 1]], axis=0)
            eu = jnp.concatenate([fill_row, e_c[:_CH - 1]], axis=0)
        else:
            up = jnp.concatenate([a_p[_CH - 1:], a_c[:_CH - 1]], axis=0)
            eu = jnp.concatenate([e_p[_CH - 1:], e_c[:_CH - 1]], axis=0)
        if k == n - 1:
            dn = jnp.concatenate([a_c[1:], a_c[_CH - 2:_CH - 1]], axis=0)
            ed = jnp.concatenate([e_c[1:], fill_row], axis=0)
        else:
            dn = jnp.concatenate([a_c[1:], a_n[:1]], axis=0)
            ed = jnp.concatenate([e_c[1:], e_n[:1]], axis=0)

        rs = up + a_c + dn
        avg = (colL(rs) + rs + colR(rs)) * (1.0 / 9.0)

        si = {0: colR(dn), 1: dn, 2: colL(dn), 3: colR(a_c),
              5: colL(a_c), 6: colR(up), 7: up}
        se = {0: colRm(ed), 1: ed, 2: colLm(ed), 3: colRm(e_c),
              5: colLm(e_c), 6: colRm(eu), 7: eu}

        if k == 0:
            ii = jax.lax.broadcasted_iota(jnp.int32, (_CH, W), 0)
            row_up = ii >= 1
        else:
            row_up = true2
        if k == n - 1:
            ii = jax.lax.broadcasted_iota(jnp.int32, (_CH, W), 0)
            row_dn = ii < _CH - 1
        else:
            row_dn = true2
        inb = {0: row_up & col_l, 1: row_up, 2: row_up & col_r, 3: col_l,
               5: col_r, 6: row_dn & col_l, 7: row_dn}

        rules = []
        for d in range(8):
            if d == 4:
                rules.append((e_c == 4, avg))
                continue
            rules.append((se[d] == d, si[d]))
            rules.append(((e_c == d) & inb[d], avg))

        def fold(seg):
            y = a_c
            any_c = None
            for c, v in seg:
                y = jnp.where(c, v, y)
                any_c = c if any_c is None else (any_c | c)
            return y, any_c

        x, _ = fold(rules[0:5])
        y2, any2 = fold(rules[5:10])
        y3, any3 = fold(rules[10:15])
        x = jnp.where(any2, y2, x)
        x = jnp.where(any3, y3, x)
        out_ref[0, k * _CH:(k + 1) * _CH, :] = x

        a_p, e_p = a_c, e_c
        a_c, e_c = a_n, e_n


@functools.partial(jax.jit, static_argnames=("interpret",))
def kernel(inp, direction, prob, interpret=False):
    B, C, H, W = inp.shape
    N = B * C
    a3 = inp.reshape(N, H, W)
    d3 = direction.reshape(N, H, W)
    p3 = prob.reshape(N, H, W)
    spec = pl.BlockSpec((1, H, W), lambda i: (i, 0, 0))
    out = pl.pallas_call(
        _body,
        grid=(N,),
        in_specs=[spec, spec, spec],
        out_specs=spec,
        out_shape=jax.ShapeDtypeStruct((N, H, W), inp.dtype),
        interpret=interpret,
    )(a3, d3, p3)
    return out.reshape(B, C, H, W)


# speedup vs baseline: 2.3161x; 1.3577x over previous
import functools

import jax
import jax.numpy as jnp
from jax.experimental import pallas as pl

_CH = 8


def _body(inp_ref, dir_ref, prob_ref, out_ref):
    G, H, W = out_ref.shape
    n = H // _CH

    jj = jax.lax.broadcasted_iota(jnp.int32, (_CH, W), 1)
    col_l, col_r = jj >= 1, jj < W - 1
    fill_col = jnp.full((_CH, 1), -1, jnp.int32)
    fill_row = jnp.full((1, W), -1, jnp.int32)
    true2 = jnp.full((_CH, W), True)

    def colL(x):
        return jnp.concatenate([x[:, 1:2], x[:, :-1]], axis=1)

    def colR(x):
        return jnp.concatenate([x[:, 1:], x[:, -2:-1]], axis=1)

    def colLm(x):
        return jnp.concatenate([fill_col, x[:, :-1]], axis=1)

    def colRm(x):
        return jnp.concatenate([x[:, 1:], fill_col], axis=1)

    for g in range(G):
        _img(inp_ref, dir_ref, prob_ref, out_ref, g, n,
             col_l, col_r, fill_col, fill_row, true2, colL, colR, colLm, colRm)


def _img(inp_ref, dir_ref, prob_ref, out_ref, g, n,
         col_l, col_r, fill_col, fill_row, true2, colL, colR, colLm, colRm):
    _CH_ = _CH
    H = out_ref.shape[1]
    W = out_ref.shape[2]

    def ld(k):
        s = slice(k * _CH, (k + 1) * _CH)
        a = inp_ref[g, s, :]
        e = jnp.where(prob_ref[g, s, :] <= 20, dir_ref[g, s, :], -1)
        return a, e

    a_p = e_p = a_n = e_n = None
    a_c, e_c = ld(0)
    for k in range(n):
        if k + 1 < n:
            a_n, e_n = ld(k + 1)
        if k == 0:
            up = jnp.concatenate([a_c[1:2], a_c[:_CH - 1]], axis=0)
            eu = jnp.concatenate([fill_row, e_c[:_CH - 1]], axis=0)
        else:
            up = jnp.concatenate([a_p[_CH - 1:], a_c[:_CH - 1]], axis=0)
            eu = jnp.concatenate([e_p[_CH - 1:], e_c[:_CH - 1]], axis=0)
        if k == n - 1:
            dn = jnp.concatenate([a_c[1:], a_c[_CH - 2:_CH - 1]], axis=0)
            ed = jnp.concatenate([e_c[1:], fill_row], axis=0)
        else:
            dn = jnp.concatenate([a_c[1:], a_n[:1]], axis=0)
            ed = jnp.concatenate([e_c[1:], e_n[:1]], axis=0)

        rs = up + a_c + dn
        avg = (colL(rs) + rs + colR(rs)) * (1.0 / 9.0)

        si = {0: colR(dn), 1: dn, 2: colL(dn), 3: colR(a_c),
              5: colL(a_c), 6: colR(up), 7: up}
        se = {0: colRm(ed), 1: ed, 2: colLm(ed), 3: colRm(e_c),
              5: colLm(e_c), 6: colRm(eu), 7: eu}

        if k == 0:
            ii = jax.lax.broadcasted_iota(jnp.int32, (_CH, W), 0)
            row_up = ii >= 1
        else:
            row_up = true2
        if k == n - 1:
            ii = jax.lax.broadcasted_iota(jnp.int32, (_CH, W), 0)
            row_dn = ii < _CH - 1
        else:
            row_dn = true2
        inb = {0: row_up & col_l, 1: row_up, 2: row_up & col_r, 3: col_l,
               5: col_r, 6: row_dn & col_l, 7: row_dn}

        x = a_c
        for d in range(8):
            if d == 4:
                x = jnp.where(e_c == 4, avg, x)
                continue
            x = jnp.where(se[d] == d, si[d], x)
            x = jnp.where((e_c == d) & inb[d], avg, x)
        out_ref[g, k * _CH:(k + 1) * _CH, :] = x

        a_p, e_p = a_c, e_c
        a_c, e_c = a_n, e_n


@functools.partial(jax.jit, static_argnames=("interpret",))
def kernel(inp, direction, prob, interpret=False):
    B, C, H, W = inp.shape
    N = B * C
    a3 = inp.reshape(N, H, W)
    d3 = direction.reshape(N, H, W)
    p3 = prob.reshape(N, H, W)
    G = 2
    spec = pl.BlockSpec((G, H, W), lambda i: (i, 0, 0))
    out = pl.pallas_call(
        _body,
        grid=(N // G,),
        in_specs=[spec, spec, spec],
        out_specs=spec,
        out_shape=jax.ShapeDtypeStruct((N, H, W), inp.dtype),
        interpret=interpret,
    )(a3, d3, p3)
    return out.reshape(B, C, H, W)


# speedup vs baseline: 2.4342x; 1.0510x over previous
import functools

import jax
import jax.numpy as jnp
from jax.experimental import pallas as pl

_CH = 8


def _body(inp_ref, dir_ref, prob_ref, out_ref):
    G, H, W = out_ref.shape
    n = H // _CH

    jj = jax.lax.broadcasted_iota(jnp.int32, (_CH, W), 1)
    col_l, col_r = jj >= 1, jj < W - 1
    fill_col = jnp.full((_CH, 1), -1, jnp.int32)
    fill_row = jnp.full((1, W), -1, jnp.int32)
    true2 = jnp.full((_CH, W), True)

    def colL(x):
        return jnp.concatenate([x[:, 1:2], x[:, :-1]], axis=1)

    def colR(x):
        return jnp.concatenate([x[:, 1:], x[:, -2:-1]], axis=1)

    def colLm(x):
        return jnp.concatenate([fill_col, x[:, :-1]], axis=1)

    def colRm(x):
        return jnp.concatenate([x[:, 1:], fill_col], axis=1)

    for g in range(G):
        _img(inp_ref, dir_ref, prob_ref, out_ref, g, n,
             col_l, col_r, fill_col, fill_row, true2, colL, colR, colLm, colRm)


def _img(inp_ref, dir_ref, prob_ref, out_ref, g, n,
         col_l, col_r, fill_col, fill_row, true2, colL, colR, colLm, colRm):
    _CH_ = _CH
    H = out_ref.shape[1]
    W = out_ref.shape[2]

    def ld(k):
        s = slice(k * _CH, (k + 1) * _CH)
        a = inp_ref[g, s, :]
        e = jnp.where(prob_ref[g, s, :] <= 20, dir_ref[g, s, :], -1)
        return a, e

    a_p = e_p = a_n = e_n = None
    a_c, e_c = ld(0)
    for k in range(n):
        if k + 1 < n:
            a_n, e_n = ld(k + 1)
        if k == 0:
            up = jnp.concatenate([a_c[1:2], a_c[:_CH - 1]], axis=0)
            eu = jnp.concatenate([fill_row, e_c[:_CH - 1]], axis=0)
        else:
            up = jnp.concatenate([a_p[_CH - 1:], a_c[:_CH - 1]], axis=0)
            eu = jnp.concatenate([e_p[_CH - 1:], e_c[:_CH - 1]], axis=0)
        if k == n - 1:
            dn = jnp.concatenate([a_c[1:], a_c[_CH - 2:_CH - 1]], axis=0)
            ed = jnp.concatenate([e_c[1:], fill_row], axis=0)
        else:
            dn = jnp.concatenate([a_c[1:], a_n[:1]], axis=0)
            ed = jnp.concatenate([e_c[1:], e_n[:1]], axis=0)

        rs = up + a_c + dn
        avg = (colL(rs) + rs + colR(rs)) * (1.0 / 9.0)

        si = {0: colR(dn), 1: dn, 2: colL(dn), 3: colR(a_c),
              5: colL(a_c), 6: colR(up), 7: up}
        se = {0: colRm(ed), 1: ed, 2: colLm(ed), 3: colRm(e_c),
              5: colLm(e_c), 6: colRm(eu), 7: eu}

        if k == 0:
            ii = jax.lax.broadcasted_iota(jnp.int32, (_CH, W), 0)
            row_up = ii >= 1
        else:
            row_up = true2
        if k == n - 1:
            ii = jax.lax.broadcasted_iota(jnp.int32, (_CH, W), 0)
            row_dn = ii < _CH - 1
        else:
            row_dn = true2
        inb = {0: row_up & col_l, 1: row_up, 2: row_up & col_r, 3: col_l,
               5: col_r, 6: row_dn & col_l, 7: row_dn}

        x = a_c
        for d in range(8):
            if d == 4:
                x = jnp.where(e_c == 4, avg, x)
                continue
            x = jnp.where(se[d] == d, si[d], x)
            x = jnp.where((e_c == d) & inb[d], avg, x)
        out_ref[g, k * _CH:(k + 1) * _CH, :] = x

        a_p, e_p = a_c, e_c
        a_c, e_c = a_n, e_n


@functools.partial(jax.jit, static_argnames=("interpret",))
def kernel(inp, direction, prob, interpret=False):
    B, C, H, W = inp.shape
    N = B * C
    a3 = inp.reshape(N, H, W)
    d3 = direction.reshape(N, H, W)
    p3 = prob.reshape(N, H, W)
    G = 4
    spec = pl.BlockSpec((G, H, W), lambda i: (i, 0, 0))
    out = pl.pallas_call(
        _body,
        grid=(N // G,),
        in_specs=[spec, spec, spec],
        out_specs=spec,
        out_shape=jax.ShapeDtypeStruct((N, H, W), inp.dtype),
        interpret=interpret,
    )(a3, d3, p3)
    return out.reshape(B, C, H, W)


# speedup vs baseline: 2.4668x; 1.0134x over previous
import functools

import jax
import jax.numpy as jnp
from jax.experimental import pallas as pl

_CH = 8


def _body(inp_ref, dir_ref, prob_ref, out_ref):
    G, H, W = out_ref.shape
    n = H // _CH

    jj = jax.lax.broadcasted_iota(jnp.int32, (_CH, W), 1)
    col_l, col_r = jj >= 1, jj < W - 1
    fill_col = jnp.full((_CH, 1), -1, jnp.int32)
    fill_row = jnp.full((1, W), -1, jnp.int32)
    true2 = jnp.full((_CH, W), True)

    def colL(x):
        return jnp.concatenate([x[:, 1:2], x[:, :-1]], axis=1)

    def colR(x):
        return jnp.concatenate([x[:, 1:], x[:, -2:-1]], axis=1)

    def colLm(x):
        return jnp.concatenate([fill_col, x[:, :-1]], axis=1)

    def colRm(x):
        return jnp.concatenate([x[:, 1:], fill_col], axis=1)

    for g in range(G):
        _img(inp_ref, dir_ref, prob_ref, out_ref, g, n,
             col_l, col_r, fill_col, fill_row, true2, colL, colR, colLm, colRm)


def _img(inp_ref, dir_ref, prob_ref, out_ref, g, n,
         col_l, col_r, fill_col, fill_row, true2, colL, colR, colLm, colRm):
    _CH_ = _CH
    H = out_ref.shape[1]
    W = out_ref.shape[2]

    def ld(k):
        s = slice(k * _CH, (k + 1) * _CH)
        a = inp_ref[g, s, :]
        e = jnp.where(prob_ref[g, s, :] <= 20, dir_ref[g, s, :], -1)
        return a, e

    a_p = e_p = a_n = e_n = None
    a_c, e_c = ld(0)
    for k in range(n):
        if k + 1 < n:
            a_n, e_n = ld(k + 1)
        if k == 0:
            up = jnp.concatenate([a_c[1:2], a_c[:_CH - 1]], axis=0)
            eu = jnp.concatenate([fill_row, e_c[:_CH - 1]], axis=0)
        else:
            up = jnp.concatenate([a_p[_CH - 1:], a_c[:_CH - 1]], axis=0)
            eu = jnp.concatenate([e_p[_CH - 1:], e_c[:_CH - 1]], axis=0)
        if k == n - 1:
            dn = jnp.concatenate([a_c[1:], a_c[_CH - 2:_CH - 1]], axis=0)
            ed = jnp.concatenate([e_c[1:], fill_row], axis=0)
        else:
            dn = jnp.concatenate([a_c[1:], a_n[:1]], axis=0)
            ed = jnp.concatenate([e_c[1:], e_n[:1]], axis=0)

        rs = up + a_c + dn
        avg = (colL(rs) + rs + colR(rs)) * (1.0 / 9.0)

        si = {0: colR(dn), 1: dn, 2: colL(dn), 3: colR(a_c),
              5: colL(a_c), 6: colR(up), 7: up}
        se = {0: colRm(ed), 1: ed, 2: colLm(ed), 3: colRm(e_c),
              5: colLm(e_c), 6: colRm(eu), 7: eu}

        if k == 0:
            ii = jax.lax.broadcasted_iota(jnp.int32, (_CH, W), 0)
            row_up = ii >= 1
        else:
            row_up = true2
        if k == n - 1:
            ii = jax.lax.broadcasted_iota(jnp.int32, (_CH, W), 0)
            row_dn = ii < _CH - 1
        else:
            row_dn = true2
        inb = {0: row_up & col_l, 1: row_up, 2: row_up & col_r, 3: col_l,
               5: col_r, 6: row_dn & col_l, 7: row_dn}

        x = a_c
        for d in range(8):
            if d == 4:
                x = jnp.where(e_c == 4, avg, x)
                continue
            x = jnp.where(se[d] == d, si[d], x)
            x = jnp.where((e_c == d) & inb[d], avg, x)
        out_ref[g, k * _CH:(k + 1) * _CH, :] = x

        a_p, e_p = a_c, e_c
        a_c, e_c = a_n, e_n


@functools.partial(jax.jit, static_argnames=("interpret",))
def kernel(inp, direction, prob, interpret=False):
    B, C, H, W = inp.shape
    N = B * C
    a3 = inp.reshape(N, H, W)
    d3 = direction.reshape(N, H, W)
    p3 = prob.reshape(N, H, W)
    G = 8
    spec = pl.BlockSpec((G, H, W), lambda i: (i, 0, 0))
    out = pl.pallas_call(
        _body,
        grid=(N // G,),
        in_specs=[spec, spec, spec],
        out_specs=spec,
        out_shape=jax.ShapeDtypeStruct((N, H, W), inp.dtype),
        interpret=interpret,
    )(a3, d3, p3)
    return out.reshape(B, C, H, W)
